# baseline (speedup 1.0000x reference)
import functools

import jax
import jax.numpy as jnp
from jax import lax
from jax.experimental import pallas as pl
from jax.experimental.pallas import tpu as pltpu

K = 8
T = 128


def _silu(x):
    return x * jax.nn.sigmoid(x)


def _router_kernel(logits_ref, idx_ref, w_ref):
    logits = logits_ref[...]
    S, E = logits.shape
    m = jnp.max(logits, axis=1, keepdims=True)
    z = jnp.exp(logits - m)
    gates = z / jnp.sum(z, axis=1, keepdims=True)

    eiota = lax.broadcasted_iota(jnp.int32, (S, E), 1)
    work = gates
    vals = []
    for k in range(K):
        mx = jnp.max(work, axis=1, keepdims=True)
        ismx = work == mx
        ck = jnp.min(jnp.where(ismx, eiota, E), axis=1, keepdims=True)
        sel = eiota == ck
        idx_ref[:, k:k + 1] = ck
        vals.append(mx)
        work = jnp.where(sel, -jnp.inf, work)
    denom = jnp.maximum(functools.reduce(jnp.add, vals),
                        jnp.finfo(jnp.float32).eps)
    for k in range(K):
        w_ref[:, k:k + 1] = vals[k] / denom


def _shared_kernel(x_ref, wgu_ref, wd_ref, out_ref):
    g = lax.dot_general(x_ref[...], wgu_ref[...], (((1,), (1,)), ((), ())),
                        preferred_element_type=jnp.float32)
    F = g.shape[1] // 2
    h = g[:, :F] * _silu(g[:, F:])
    out_ref[...] = lax.dot_general(h, wd_ref[...], (((1,), (1,)), ((), ())),
                                   preferred_element_type=jnp.float32)


def _expert_kernel(se_ref, base_ref, valid_ref,
                   x_ref, idx_ref, prio_ref, w_ref, wgu_ref, wd_ref, out_ref):
    c = pl.program_id(0)
    i = pl.program_id(1)

    @pl.when((c == 0) & (i == 0))
    def _init():
        out_ref[...] = jnp.zeros_like(out_ref)

    @pl.when(valid_ref[i] == 1)
    def _compute():
        e = se_ref[i]
        base = base_ref[i]
        S = x_ref.shape[0]
        slot_iota = lax.broadcasted_iota(jnp.int32, (S, T), 1)
        M = jnp.zeros((S, T), jnp.float32)
        Mw = jnp.zeros((S, T), jnp.float32)
        for k in range(K):
            idx_c = idx_ref[:, k:k + 1]
            prio_c = prio_ref[:, k:k + 1]
            w_c = w_ref[:, k:k + 1]
            cond = (idx_c == e) & ((prio_c - base) == slot_iota)
            M = M + cond.astype(jnp.float32)
            Mw = Mw + jnp.where(cond, w_c, 0.0)
        xd = lax.dot_general(M, x_ref[...], (((0,), (0,)), ((), ())),
                             preferred_element_type=jnp.float32)
        g1 = lax.dot_general(xd, wgu_ref[0, 0], (((1,), (1,)), ((), ())),
                             preferred_element_type=jnp.float32)
        g2 = lax.dot_general(xd, wgu_ref[0, 1], (((1,), (1,)), ((), ())),
                             preferred_element_type=jnp.float32)
        h = g1 * _silu(g2)
        y = lax.dot_general(h, wd_ref[0], (((1,), (1,)), ((), ())),
                            preferred_element_type=jnp.float32)
        out_ref[...] += lax.dot_general(Mw, y, (((1,), (0,)), ((), ())),
                                        preferred_element_type=jnp.float32)


def kernel(hidden_states, W_shared_gu, W_shared_down, W_gate,
           W_experts_gu, W_experts_down):
    B, S, H = hidden_states.shape
    E, F2, _ = W_experts_gu.shape
    F = F2 // 2
    x = hidden_states.reshape(S, H).astype(jnp.float32)

    logits = x @ W_gate.T
    idx8, w8 = pl.pallas_call(
        _router_kernel,
        out_shape=(jax.ShapeDtypeStruct((S, K), jnp.int32),
                   jax.ShapeDtypeStruct((S, K), jnp.float32)),
    )(logits)

    pairs = idx8.reshape(-1)
    em = (pairs[:, None] == jnp.arange(E, dtype=jnp.int32)[None, :]
          ).astype(jnp.int32)
    c = jnp.cumsum(em, axis=0)
    prio8 = (jnp.sum(c * em, axis=1) - 1).reshape(S, K).astype(jnp.int32)
    counts = c[-1]

    tpe = (counts + T - 1) // T
    tile_cum = jnp.cumsum(tpe)
    tstart = tile_cum - tpe
    total_tiles = tile_cum[-1]
    NT = S * K // T + E
    j = jnp.arange(NT, dtype=jnp.int32)
    expert_of = jnp.sum((j[:, None] >= tile_cum[None, :]).astype(jnp.int32),
                        axis=1)
    valid = (j < total_tiles).astype(jnp.int32)
    last_e = jnp.max(jnp.where(counts > 0, jnp.arange(E, dtype=jnp.int32), -1))
    tile_expert = jnp.where(valid == 1, jnp.minimum(expert_of, E - 1), last_e
                            ).astype(jnp.int32)
    tile_base = ((j - jnp.take(tstart, tile_expert)) * T).astype(jnp.int32)

    NC = 3
    Fc = F // NC
    wgu4 = W_experts_gu.reshape(E, 2, F, H)
    grid_spec = pltpu.PrefetchScalarGridSpec(
        num_scalar_prefetch=3,
        grid=(NC, NT),
        in_specs=[
            pl.BlockSpec((S, H), lambda c, i, se, b, v: (0, 0)),
            pl.BlockSpec((S, K), lambda c, i, se, b, v: (0, 0)),
            pl.BlockSpec((S, K), lambda c, i, se, b, v: (0, 0)),
            pl.BlockSpec((S, K), lambda c, i, se, b, v: (0, 0)),
            pl.BlockSpec((1, 2, Fc, H), lambda c, i, se, b, v: (se[i], 0, c, 0)),
            pl.BlockSpec((1, H, Fc), lambda c, i, se, b, v: (se[i], 0, c)),
        ],
        out_specs=pl.BlockSpec((S, H), lambda c, i, se, b, v: (0, 0)),
    )
    combined = pl.pallas_call(
        _expert_kernel,
        grid_spec=grid_spec,
        out_shape=jax.ShapeDtypeStruct((S, H), jnp.float32),
        compiler_params=pltpu.CompilerParams(
            dimension_semantics=("arbitrary", "arbitrary")),
    )(tile_expert, tile_base, valid, x, idx8, prio8, w8,
      wgu4, W_experts_down)

    TS = 128
    shared = pl.pallas_call(
        _shared_kernel,
        grid=(S // TS,),
        in_specs=[
            pl.BlockSpec((TS, H), lambda i: (i, 0)),
            pl.BlockSpec((F2, H), lambda i: (0, 0)),
            pl.BlockSpec((H, F), lambda i: (0, 0)),
        ],
        out_specs=pl.BlockSpec((TS, H), lambda i: (i, 0)),
        out_shape=jax.ShapeDtypeStruct((S, H), jnp.float32),
    )(x, W_shared_gu, W_shared_down)

    return (shared + combined).reshape(B, S, H)


# speedup vs baseline: 1.6235x; 1.6235x over previous
import functools

import jax
import jax.numpy as jnp
from jax import lax
from jax.experimental import pallas as pl
from jax.experimental.pallas import tpu as pltpu

K = 8
T = 128


def _silu(x):
    return x * jax.nn.sigmoid(x)


def _router_kernel(logits_ref, idx_ref, w_ref):
    logits = logits_ref[...]
    S, E = logits.shape
    m = jnp.max(logits, axis=1, keepdims=True)
    z = jnp.exp(logits - m)
    gates = z / jnp.sum(z, axis=1, keepdims=True)

    eiota = lax.broadcasted_iota(jnp.int32, (S, E), 1)
    work = gates
    vals = []
    for k in range(K):
        mx = jnp.max(work, axis=1, keepdims=True)
        ismx = work == mx
        ck = jnp.min(jnp.where(ismx, eiota, E), axis=1, keepdims=True)
        sel = eiota == ck
        idx_ref[:, k:k + 1] = ck
        vals.append(mx)
        work = jnp.where(sel, -jnp.inf, work)
    denom = jnp.maximum(functools.reduce(jnp.add, vals),
                        jnp.finfo(jnp.float32).eps)
    for k in range(K):
        w_ref[:, k:k + 1] = vals[k] / denom


def _shared_kernel(x_ref, wgu_ref, wd_ref, out_ref):
    g = lax.dot_general(x_ref[...], wgu_ref[...], (((1,), (1,)), ((), ())),
                        preferred_element_type=jnp.float32)
    F = g.shape[1] // 2
    h = g[:, :F] * _silu(g[:, F:])
    out_ref[...] = lax.dot_general(h, wd_ref[...], (((1,), (1,)), ((), ())),
                                   preferred_element_type=jnp.float32)


def _expert_kernel(se_ref, valid_ref,
                   x_ref, pk_ref, wgu_ref, wd_ref, out_ref):
    c = pl.program_id(0)
    i = pl.program_id(1)

    @pl.when((c == 0) & (i == 0))
    def _init():
        out_ref[...] = jnp.zeros_like(out_ref)

    @pl.when(valid_ref[i] == 1)
    def _compute():
        S = x_ref.shape[0]
        lo = i * T
        hit = jnp.zeros((S, 1), jnp.bool_)
        rsum = jnp.zeros((S, 1), jnp.int32)
        wsel = jnp.zeros((S, 1), jnp.float32)
        for k in range(K):
            s_c = pk_ref[:, k:k + 1]
            w_c = lax.bitcast_convert_type(pk_ref[:, K + k:K + k + 1],
                                           jnp.float32)
            m = (s_c >= lo) & (s_c < lo + T)
            hit = hit | m
            rsum = rsum + jnp.where(m, s_c, 0)
            wsel = wsel + jnp.where(m, w_c, 0.0)
        r = jnp.where(hit, rsum - lo, -1)
        slot_iota = lax.broadcasted_iota(jnp.int32, (S, T), 1)
        M = (slot_iota == r).astype(jnp.float32)
        Mw = M * wsel
        xd = lax.dot_general(M, x_ref[...], (((0,), (0,)), ((), ())),
                             preferred_element_type=jnp.float32)
        g1 = lax.dot_general(xd, wgu_ref[0, 0], (((1,), (1,)), ((), ())),
                             preferred_element_type=jnp.float32)
        g2 = lax.dot_general(xd, wgu_ref[0, 1], (((1,), (1,)), ((), ())),
                             preferred_element_type=jnp.float32)
        h = g1 * _silu(g2)
        y = lax.dot_general(h, wd_ref[0], (((1,), (1,)), ((), ())),
                            preferred_element_type=jnp.float32)
        out_ref[...] += lax.dot_general(Mw, y, (((1,), (0,)), ((), ())),
                                        preferred_element_type=jnp.float32)


def kernel(hidden_states, W_shared_gu, W_shared_down, W_gate,
           W_experts_gu, W_experts_down):
    B, S, H = hidden_states.shape
    E, F2, _ = W_experts_gu.shape
    F = F2 // 2
    x = hidden_states.reshape(S, H).astype(jnp.float32)

    logits = x @ W_gate.T
    idx8, w8 = pl.pallas_call(
        _router_kernel,
        out_shape=(jax.ShapeDtypeStruct((S, K), jnp.int32),
                   jax.ShapeDtypeStruct((S, K), jnp.float32)),
    )(logits)

    pairs = idx8.reshape(-1)
    em = (pairs[:, None] == jnp.arange(E, dtype=jnp.int32)[None, :]
          ).astype(jnp.int32)
    c = jnp.cumsum(em, axis=0)
    prio8 = (jnp.sum(c * em, axis=1) - 1).reshape(S, K).astype(jnp.int32)
    counts = c[-1]

    tpe = (counts + T - 1) // T
    tile_cum = jnp.cumsum(tpe)
    tstart = tile_cum - tpe
    total_tiles = tile_cum[-1]
    NT = S * K // T + E
    j = jnp.arange(NT, dtype=jnp.int32)
    expert_of = jnp.sum((j[:, None] >= tile_cum[None, :]).astype(jnp.int32),
                        axis=1)
    valid = (j < total_tiles).astype(jnp.int32)
    last_e = jnp.max(jnp.where(counts > 0, jnp.arange(E, dtype=jnp.int32), -1))
    tile_expert = jnp.where(valid == 1, jnp.minimum(expert_of, E - 1), last_e
                            ).astype(jnp.int32)
    slot8 = (jnp.take(tstart * T, idx8) + prio8).astype(jnp.int32)
    packed = jnp.concatenate(
        [slot8, lax.bitcast_convert_type(w8, jnp.int32)], axis=1)

    NC = 2
    Fc = F // NC
    wgu4 = W_experts_gu.reshape(E, 2, F, H)
    grid_spec = pltpu.PrefetchScalarGridSpec(
        num_scalar_prefetch=2,
        grid=(NC, NT),
        in_specs=[
            pl.BlockSpec((S, H), lambda c, i, se, v: (0, 0)),
            pl.BlockSpec((S, 2 * K), lambda c, i, se, v: (0, 0)),
            pl.BlockSpec((1, 2, Fc, H), lambda c, i, se, v: (se[i], 0, c, 0)),
            pl.BlockSpec((1, H, Fc), lambda c, i, se, v: (se[i], 0, c)),
        ],
        out_specs=pl.BlockSpec((S, H), lambda c, i, se, v: (0, 0)),
    )
    combined = pl.pallas_call(
        _expert_kernel,
        grid_spec=grid_spec,
        out_shape=jax.ShapeDtypeStruct((S, H), jnp.float32),
        compiler_params=pltpu.CompilerParams(
            dimension_semantics=("arbitrary", "arbitrary")),
    )(tile_expert, valid, x, packed, wgu4, W_experts_down)

    TS = 128
    shared = pl.pallas_call(
        _shared_kernel,
        grid=(S // TS,),
        in_specs=[
            pl.BlockSpec((TS, H), lambda i: (i, 0)),
            pl.BlockSpec((F2, H), lambda i: (0, 0)),
            pl.BlockSpec((H, F), lambda i: (0, 0)),
        ],
        out_specs=pl.BlockSpec((TS, H), lambda i: (i, 0)),
        out_shape=jax.ShapeDtypeStruct((S, H), jnp.float32),
    )(x, W_shared_gu, W_shared_down)

    return (shared + combined).reshape(B, S, H)


# speedup vs baseline: 3.2563x; 2.0057x over previous
import functools

import jax
import jax.numpy as jnp
from jax import lax
from jax.experimental import pallas as pl
from jax.experimental.pallas import tpu as pltpu

K = 8
T = 128


def _silu(x):
    return x * jax.nn.sigmoid(x)


def _router_kernel(logits_ref, idx_ref, w_ref):
    logits = logits_ref[...]
    S, E = logits.shape
    m = jnp.max(logits, axis=1, keepdims=True)
    z = jnp.exp(logits - m)
    gates = z / jnp.sum(z, axis=1, keepdims=True)

    eiota = lax.broadcasted_iota(jnp.int32, (S, E), 1)
    work = gates
    vals = []
    for k in range(K):
        mx = jnp.max(work, axis=1, keepdims=True)
        ismx = work == mx
        ck = jnp.min(jnp.where(ismx, eiota, E), axis=1, keepdims=True)
        sel = eiota == ck
        idx_ref[:, k:k + 1] = ck
        vals.append(mx)
        work = jnp.where(sel, -jnp.inf, work)
    denom = jnp.maximum(functools.reduce(jnp.add, vals),
                        jnp.finfo(jnp.float32).eps)
    for k in range(K):
        w_ref[:, k:k + 1] = vals[k] / denom


def _shared_kernel(x_ref, wgu_ref, wd_ref, out_ref):
    g = lax.dot_general(x_ref[...], wgu_ref[...], (((1,), (1,)), ((), ())),
                        preferred_element_type=jnp.float32)
    F = g.shape[1] // 2
    h = g[:, :F] * _silu(g[:, F:])
    out_ref[...] = lax.dot_general(h, wd_ref[...], (((1,), (1,)), ((), ())),
                                   preferred_element_type=jnp.float32)


def _expert_kernel(se_ref, valid_ref,
                   x_ref, pk_ref, wgu_ref, wd_ref, out_ref):
    c = pl.program_id(0)
    i = pl.program_id(1)

    @pl.when((c == 0) & (i == 0))
    def _init():
        out_ref[...] = jnp.zeros_like(out_ref)

    @pl.when(valid_ref[i] == 1)
    def _compute():
        S = x_ref.shape[0]
        lo = i * T
        hit = jnp.zeros((1, S), jnp.bool_)
        rsum = jnp.zeros((1, S), jnp.int32)
        wsel = jnp.zeros((1, S), jnp.float32)
        for k in range(K):
            s_r = pk_ref[k:k + 1, :]
            w_r = lax.bitcast_convert_type(pk_ref[K + k:K + k + 1, :],
                                           jnp.float32)
            m = (s_r >= lo) & (s_r < lo + T)
            hit = hit | m
            rsum = rsum + jnp.where(m, s_r, 0)
            wsel = wsel + jnp.where(m, w_r, 0.0)
        r = jnp.where(hit, rsum - lo, -1)
        slot_iota = lax.broadcasted_iota(jnp.int32, (T, S), 0)
        M = (slot_iota == r).astype(jnp.float32)
        Mw = M * wsel
        xd = lax.dot_general(M, x_ref[...], (((1,), (0,)), ((), ())),
                             preferred_element_type=jnp.float32)
        g1 = lax.dot_general(xd, wgu_ref[0, 0], (((1,), (1,)), ((), ())),
                             preferred_element_type=jnp.float32)
        g2 = lax.dot_general(xd, wgu_ref[0, 1], (((1,), (1,)), ((), ())),
                             preferred_element_type=jnp.float32)
        h = g1 * _silu(g2)
        y = lax.dot_general(h, wd_ref[0], (((1,), (1,)), ((), ())),
                            preferred_element_type=jnp.float32)
        out_ref[...] += lax.dot_general(Mw, y, (((0,), (0,)), ((), ())),
                                        preferred_element_type=jnp.float32)


def kernel(hidden_states, W_shared_gu, W_shared_down, W_gate,
           W_experts_gu, W_experts_down):
    B, S, H = hidden_states.shape
    E, F2, _ = W_experts_gu.shape
    F = F2 // 2
    x = hidden_states.reshape(S, H).astype(jnp.float32)

    logits = x @ W_gate.T
    idx8, w8 = pl.pallas_call(
        _router_kernel,
        out_shape=(jax.ShapeDtypeStruct((S, K), jnp.int32),
                   jax.ShapeDtypeStruct((S, K), jnp.float32)),
    )(logits)

    pairs = idx8.reshape(-1)
    em = (pairs[:, None] == jnp.arange(E, dtype=jnp.int32)[None, :]
          ).astype(jnp.int32)
    c = jnp.cumsum(em, axis=0)
    prio8 = (jnp.sum(c * em, axis=1) - 1).reshape(S, K).astype(jnp.int32)
    counts = c[-1]

    tpe = (counts + T - 1) // T
    tile_cum = jnp.cumsum(tpe)
    tstart = tile_cum - tpe
    total_tiles = tile_cum[-1]
    NT = S * K // T + E
    j = jnp.arange(NT, dtype=jnp.int32)
    expert_of = jnp.sum((j[:, None] >= tile_cum[None, :]).astype(jnp.int32),
                        axis=1)
    valid = (j < total_tiles).astype(jnp.int32)
    last_e = jnp.max(jnp.where(counts > 0, jnp.arange(E, dtype=jnp.int32), -1))
    tile_expert = jnp.where(valid == 1, jnp.minimum(expert_of, E - 1), last_e
                            ).astype(jnp.int32)
    slot8 = (jnp.take(tstart * T, idx8) + prio8).astype(jnp.int32)
    packed = jnp.concatenate(
        [slot8, lax.bitcast_convert_type(w8, jnp.int32)], axis=1).T

    NC = 2
    Fc = F // NC
    wgu4 = W_experts_gu.reshape(E, 2, F, H)
    grid_spec = pltpu.PrefetchScalarGridSpec(
        num_scalar_prefetch=2,
        grid=(NC, NT),
        in_specs=[
            pl.BlockSpec((S, H), lambda c, i, se, v: (0, 0)),
            pl.BlockSpec((2 * K, S), lambda c, i, se, v: (0, 0)),
            pl.BlockSpec((1, 2, Fc, H), lambda c, i, se, v: (se[i], 0, c, 0)),
            pl.BlockSpec((1, H, Fc), lambda c, i, se, v: (se[i], 0, c)),
        ],
        out_specs=pl.BlockSpec((S, H), lambda c, i, se, v: (0, 0)),
    )
    combined = pl.pallas_call(
        _expert_kernel,
        grid_spec=grid_spec,
        out_shape=jax.ShapeDtypeStruct((S, H), jnp.float32),
        compiler_params=pltpu.CompilerParams(
            dimension_semantics=("arbitrary", "arbitrary")),
    )(tile_expert, valid, x, packed, wgu4, W_experts_down)

    TS = 128
    shared = pl.pallas_call(
        _shared_kernel,
        grid=(S // TS,),
        in_specs=[
            pl.BlockSpec((TS, H), lambda i: (i, 0)),
            pl.BlockSpec((F2, H), lambda i: (0, 0)),
            pl.BlockSpec((H, F), lambda i: (0, 0)),
        ],
        out_specs=pl.BlockSpec((TS, H), lambda i: (i, 0)),
        out_shape=jax.ShapeDtypeStruct((S, H), jnp.float32),
    )(x, W_shared_gu, W_shared_down)

    return (shared + combined).reshape(B, S, H)


# speedup vs baseline: 3.2781x; 1.0067x over previous
import functools

import jax
import jax.numpy as jnp
from jax import lax
from jax.experimental import pallas as pl
from jax.experimental.pallas import tpu as pltpu

K = 8
T = 128


def _silu(x):
    return x * jax.nn.sigmoid(x)


def _router_kernel(logits_ref, idx_ref, w_ref):
    logits = logits_ref[...]
    S, E = logits.shape
    m = jnp.max(logits, axis=1, keepdims=True)
    z = jnp.exp(logits - m)
    gates = z / jnp.sum(z, axis=1, keepdims=True)

    eiota = lax.broadcasted_iota(jnp.int32, (S, E), 1)
    work = gates
    vals = []
    for k in range(K):
        mx = jnp.max(work, axis=1, keepdims=True)
        ismx = work == mx
        ck = jnp.min(jnp.where(ismx, eiota, E), axis=1, keepdims=True)
        sel = eiota == ck
        idx_ref[:, k:k + 1] = ck
        vals.append(mx)
        work = jnp.where(sel, -jnp.inf, work)
    denom = jnp.maximum(functools.reduce(jnp.add, vals),
                        jnp.finfo(jnp.float32).eps)
    for k in range(K):
        w_ref[:, k:k + 1] = vals[k] / denom


def _shared_kernel(x_ref, wgu_ref, wd_ref, out_ref):
    g = lax.dot_general(x_ref[...], wgu_ref[...], (((1,), (1,)), ((), ())),
                        preferred_element_type=jnp.float32)
    F = g.shape[1] // 2
    h = g[:, :F] * _silu(g[:, F:])
    out_ref[...] = lax.dot_general(h, wd_ref[...], (((1,), (1,)), ((), ())),
                                   preferred_element_type=jnp.float32)


def _expert_kernel(se_ref, valid_ref,
                   x_ref, pk_ref, wgu_ref, wd_ref, out_ref):
    c = pl.program_id(0)
    i = pl.program_id(1)

    @pl.when((c == 0) & (i == 0))
    def _init():
        out_ref[...] = jnp.zeros_like(out_ref)

    @pl.when(valid_ref[i] == 1)
    def _compute():
        S = x_ref.shape[0]
        lo = i * T
        hit = jnp.zeros((1, S), jnp.bool_)
        rsum = jnp.zeros((1, S), jnp.int32)
        wsel = jnp.zeros((1, S), jnp.float32)
        for k in range(K):
            s_r = pk_ref[k:k + 1, :]
            w_r = lax.bitcast_convert_type(pk_ref[K + k:K + k + 1, :],
                                           jnp.float32)
            m = (s_r >= lo) & (s_r < lo + T)
            hit = hit | m
            rsum = rsum + jnp.where(m, s_r, 0)
            wsel = wsel + jnp.where(m, w_r, 0.0)
        r = jnp.where(hit, rsum - lo, -1)
        slot_iota = lax.broadcasted_iota(jnp.int32, (T, S), 0)
        M = (slot_iota == r).astype(jnp.bfloat16)
        Mw = M * wsel.astype(jnp.bfloat16)
        xd = lax.dot_general(M, x_ref[...], (((1,), (0,)), ((), ())),
                             preferred_element_type=jnp.float32)
        g1 = lax.dot_general(xd, wgu_ref[0, 0], (((1,), (1,)), ((), ())),
                             preferred_element_type=jnp.float32)
        g2 = lax.dot_general(xd, wgu_ref[0, 1], (((1,), (1,)), ((), ())),
                             preferred_element_type=jnp.float32)
        h = g1 * _silu(g2)
        y = lax.dot_general(h, wd_ref[0], (((1,), (1,)), ((), ())),
                            preferred_element_type=jnp.float32)
        out_ref[...] += lax.dot_general(Mw, y.astype(jnp.bfloat16),
                                        (((0,), (0,)), ((), ())),
                                        preferred_element_type=jnp.float32)


def kernel(hidden_states, W_shared_gu, W_shared_down, W_gate,
           W_experts_gu, W_experts_down):
    B, S, H = hidden_states.shape
    E, F2, _ = W_experts_gu.shape
    F = F2 // 2
    x = hidden_states.reshape(S, H).astype(jnp.float32)

    logits = x @ W_gate.T
    idx8, w8 = pl.pallas_call(
        _router_kernel,
        out_shape=(jax.ShapeDtypeStruct((S, K), jnp.int32),
                   jax.ShapeDtypeStruct((S, K), jnp.float32)),
    )(logits)

    pairs = idx8.reshape(-1)
    em = (pairs[:, None] == jnp.arange(E, dtype=jnp.int32)[None, :]
          ).astype(jnp.int32)
    c = jnp.cumsum(em, axis=0)
    prio8 = (jnp.sum(c * em, axis=1) - 1).reshape(S, K).astype(jnp.int32)
    counts = c[-1]

    tpe = (counts + T - 1) // T
    tile_cum = jnp.cumsum(tpe)
    tstart = tile_cum - tpe
    total_tiles = tile_cum[-1]
    NT = S * K // T + E
    j = jnp.arange(NT, dtype=jnp.int32)
    expert_of = jnp.sum((j[:, None] >= tile_cum[None, :]).astype(jnp.int32),
                        axis=1)
    valid = (j < total_tiles).astype(jnp.int32)
    last_e = jnp.max(jnp.where(counts > 0, jnp.arange(E, dtype=jnp.int32), -1))
    tile_expert = jnp.where(valid == 1, jnp.minimum(expert_of, E - 1), last_e
                            ).astype(jnp.int32)
    slot8 = (jnp.take(tstart * T, idx8) + prio8).astype(jnp.int32)
    packed = jnp.concatenate(
        [slot8, lax.bitcast_convert_type(w8, jnp.int32)], axis=1).T

    NC = 2
    Fc = F // NC
    wgu4 = W_experts_gu.reshape(E, 2, F, H)
    grid_spec = pltpu.PrefetchScalarGridSpec(
        num_scalar_prefetch=2,
        grid=(NC, NT),
        in_specs=[
            pl.BlockSpec((S, H), lambda c, i, se, v: (0, 0)),
            pl.BlockSpec((2 * K, S), lambda c, i, se, v: (0, 0)),
            pl.BlockSpec((1, 2, Fc, H), lambda c, i, se, v: (se[i], 0, c, 0)),
            pl.BlockSpec((1, H, Fc), lambda c, i, se, v: (se[i], 0, c)),
        ],
        out_specs=pl.BlockSpec((S, H), lambda c, i, se, v: (0, 0)),
    )
    combined = pl.pallas_call(
        _expert_kernel,
        grid_spec=grid_spec,
        out_shape=jax.ShapeDtypeStruct((S, H), jnp.float32),
        compiler_params=pltpu.CompilerParams(
            dimension_semantics=("arbitrary", "arbitrary")),
    )(tile_expert, valid, x.astype(jnp.bfloat16), packed, wgu4,
      W_experts_down)

    TS = 128
    shared = pl.pallas_call(
        _shared_kernel,
        grid=(S // TS,),
        in_specs=[
            pl.BlockSpec((TS, H), lambda i: (i, 0)),
            pl.BlockSpec((F2, H), lambda i: (0, 0)),
            pl.BlockSpec((H, F), lambda i: (0, 0)),
        ],
        out_specs=pl.BlockSpec((TS, H), lambda i: (i, 0)),
        out_shape=jax.ShapeDtypeStruct((S, H), jnp.float32),
    )(x, W_shared_gu, W_shared_down)

    return (shared + combined).reshape(B, S, H)
